# baseline (speedup 1.0000x reference)
import jax
import jax.numpy as jnp
import numpy as np
from jax.experimental import pallas as pl
from jax.experimental.pallas import tpu as pltpu

_B = 2
_N = 96
_F = 16
_C = 3


def _interp_matrix():
    x = np.linspace(1.0, float(_F) - 2.0, _N).astype(np.float32)
    f = np.floor(x)
    i0 = np.clip(f.astype(np.int64), 0, _F - 1)
    i1 = np.clip(f.astype(np.int64) + 1, 0, _F - 1)
    w = (x - f).astype(np.float32)
    a = np.zeros((_N, _F), dtype=np.float32)
    np.add.at(a, (np.arange(_N), i0), 1.0 - w)
    np.add.at(a, (np.arange(_N), i1), w)
    return a


_A = _interp_matrix()
_PREC = jax.lax.Precision.HIGHEST


def _taps():
    x = np.linspace(1.0, float(_F) - 2.0, _N).astype(np.float32)
    f = np.floor(x)
    i0 = np.clip(f.astype(np.int64), 0, _F - 1)
    i1 = np.clip(f.astype(np.int64) + 1, 0, _F - 1)
    w = (x - f).astype(np.float32)
    return [(int(a), int(b), float(ww)) for a, b, ww in zip(i0, i1, w)]


_TAPS = _taps()


def _warp_kernel(f_ref, azt_ref, ay_ref, o_ref, t_ref):
    fbc = f_ref[0, :, 0].reshape(_F * _F, _F)
    v = jnp.dot(fbc, azt_ref[...], precision=_PREC,
                preferred_element_type=jnp.float32)
    for i in range(_F):
        t_ref[i] = jnp.dot(ay_ref[...], v[i * _F:(i + 1) * _F],
                           precision=_PREC,
                           preferred_element_type=jnp.float32)
    for x, (n0, n1, w) in enumerate(_TAPS):
        o_ref[0, x, 0] = (1.0 - w) * t_ref[n0] + w * t_ref[n1]


@jax.jit
def kernel(field):
    ft = jnp.transpose(field, (0, 1, 4, 2, 3))
    out = pl.pallas_call(
        _warp_kernel,
        grid=(_B, _C),
        in_specs=[
            pl.BlockSpec((1, _F, 1, _F, _F), lambda b, c: (b, 0, c, 0, 0)),
            pl.BlockSpec((_F, _N), lambda b, c: (0, 0)),
            pl.BlockSpec((_N, _F), lambda b, c: (0, 0)),
        ],
        out_specs=pl.BlockSpec((1, _N, 1, _N, _N),
                               lambda b, c: (b, 0, c, 0, 0)),
        out_shape=jax.ShapeDtypeStruct((_B, _N, _C, _N, _N), jnp.float32),
        scratch_shapes=[pltpu.VMEM((_F, _N, _N), jnp.float32)],
        compiler_params=pltpu.CompilerParams(
            dimension_semantics=("parallel", "parallel")),
    )(ft, jnp.asarray(_A.T), jnp.asarray(_A))
    return jnp.transpose(out, (0, 1, 3, 4, 2))
